# baseline (speedup 1.0000x reference)
import functools

import jax
import jax.numpy as jnp
from jax.experimental import pallas as pl
from jax.experimental.pallas import tpu as pltpu

_N = 64
_K = 8
_ALPHA = 0.001


def _router_body(x_ref, w_ref, probs_ref, ids_ref, wts_ref, aux_ref, usage_ref,
                 cnt_ref, ps_ref, *, nt, tokens):
    i = pl.program_id(0)
    logits = jnp.dot(x_ref[...], w_ref[...], preferred_element_type=jnp.float32)
    m = jnp.max(logits, axis=-1, keepdims=True)
    e = jnp.exp(logits - m)
    p = e / jnp.sum(e, axis=-1, keepdims=True)
    probs_ref[...] = p

    tb = logits.shape[0]
    work = p.T
    iota0 = jax.lax.broadcasted_iota(jnp.int32, (_N, tb), 0)
    kiota0 = jax.lax.broadcasted_iota(jnp.int32, (_K, tb), 0)
    ids_t = jnp.zeros((_K, tb), jnp.int32)
    wts_t = jnp.zeros((_K, tb), jnp.float32)
    for k in range(_K):
        mv = jnp.max(work, axis=0, keepdims=True)
        im = jnp.min(jnp.where(work == mv, iota0, _N), axis=0, keepdims=True)
        ids_t = jnp.where(kiota0 == k, im, ids_t)
        wts_t = jnp.where(kiota0 == k, mv, wts_t)
        work = jnp.where(iota0 == im, -1.0, work)
    ids_ref[...] = ids_t.T
    wts_ref[...] = (wts_t / jnp.sum(wts_t, axis=0, keepdims=True)).T

    blk_cnt = jnp.sum(jnp.where(work < 0.0, 1.0, 0.0), axis=1).reshape(1, _N)
    blk_ps = jnp.sum(p, axis=0, keepdims=True)

    @pl.when(i == 0)
    def _():
        cnt_ref[...] = blk_cnt
        ps_ref[...] = blk_ps

    @pl.when(i != 0)
    def _():
        cnt_ref[...] += blk_cnt
        ps_ref[...] += blk_ps

    @pl.when(i == nt - 1)
    def _():
        cnt = cnt_ref[...]
        f_i = cnt / float(tokens * _K)
        p_i = ps_ref[...] / float(tokens)
        aux_ref[...] = (_ALPHA * _N) * jnp.sum(f_i * p_i, axis=1, keepdims=True)
        usage_ref[...] = cnt / jnp.sum(cnt, axis=1, keepdims=True)


def kernel(hidden_states, gate_weight):
    b, l, d = hidden_states.shape
    n = gate_weight.shape[0]
    tokens = b * l
    x = hidden_states.reshape(tokens, d)
    wt = gate_weight.T

    tb = 1024
    nt = tokens // tb

    body = functools.partial(_router_body, nt=nt, tokens=tokens)
    probs, ids, wts, aux, usage = pl.pallas_call(
        body,
        grid=(nt,),
        in_specs=[
            pl.BlockSpec((tb, d), lambda i: (i, 0)),
            pl.BlockSpec((d, n), lambda i: (0, 0)),
        ],
        out_specs=[
            pl.BlockSpec((tb, n), lambda i: (i, 0)),
            pl.BlockSpec((tb, _K), lambda i: (i, 0)),
            pl.BlockSpec((tb, _K), lambda i: (i, 0)),
            pl.BlockSpec((1, 1), lambda i: (0, 0)),
            pl.BlockSpec((1, n), lambda i: (0, 0)),
        ],
        out_shape=[
            jax.ShapeDtypeStruct((tokens, n), jnp.float32),
            jax.ShapeDtypeStruct((tokens, _K), jnp.int32),
            jax.ShapeDtypeStruct((tokens, _K), jnp.float32),
            jax.ShapeDtypeStruct((1, 1), jnp.float32),
            jax.ShapeDtypeStruct((1, n), jnp.float32),
        ],
        scratch_shapes=[
            pltpu.VMEM((1, n), jnp.float32),
            pltpu.VMEM((1, n), jnp.float32),
        ],
    )(x, wt)

    return (probs.reshape(b, l, n),
            ids.reshape(b, l, _K),
            wts.reshape(b, l, _K),
            aux[0, 0],
            usage[0])
